# Initial kernel scaffold; baseline (speedup 1.0000x reference)
import jax
import jax.numpy as jnp
from jax.experimental import pallas as pl


def kernel(z, code):
    raise NotImplementedError("write your pallas kernel here")


# speedup vs baseline: 1.0286x; 1.0286x over previous
import functools

import jax
import jax.numpy as jnp
from jax import lax
from jax.experimental import pallas as pl
from jax.experimental.pallas import tpu as pltpu
from jax.experimental.pallas import tpu_sc as plsc

N_TOK = 16384
NE = 8192
D = 256
BM = 128
NB = N_TOK // BM
BMC = 512
NBC = N_TOK // BMC

SC_NC = 2
SC_NS = 16
SC_NW = SC_NC * SC_NS
SC_BPW = N_TOK // SC_NW
SC_CH = 128
SC_NCH = SC_BPW // SC_CH


def _argmin_body(z_ref, code_ref, idx_ref, cnt_scr, c2_scr):
    i = pl.program_id(0)

    @pl.when(i == 0)
    def _init():
        c = code_ref[...]
        n = jnp.sqrt(jnp.sum(c * c, axis=1, keepdims=True))
        cn = c / jnp.maximum(n, 1e-12)
        cnt_scr[...] = cn.T
        cnt = cnt_scr[...]
        c2_scr[...] = jnp.sum(cnt * cnt, axis=0, keepdims=True)

    z = z_ref[...]
    zn = z / jnp.maximum(jnp.sqrt(jnp.sum(z * z, axis=1, keepdims=True)), 1e-12)
    z2n = jnp.sum(zn * zn, axis=1, keepdims=True)
    s2 = lax.dot_general(zn * (-2.0), cnt_scr[...], (((1,), (0,)), ((), ())),
                         preferred_element_type=jnp.float32)
    t = (z2n + c2_scr[...]) + s2
    m = jnp.min(t, axis=1, keepdims=True)
    iota = lax.broadcasted_iota(jnp.int32, (BM, NE), 1)
    idx = jnp.min(jnp.where(t == m, iota, jnp.int32(2 ** 30)), axis=1)
    idx_ref[0, 0, :] = idx


def _finalize_body(z_ref, g_ref, zq_ref, loss_ref):
    i = pl.program_id(0)
    g = g_ref[...]
    n = jnp.sqrt(jnp.sum(g * g, axis=1, keepdims=True))
    zq = g / jnp.maximum(n, 1e-12)
    zq_ref[...] = zq
    dlt = zq - z_ref[...]
    ssq = jnp.sum(dlt * dlt, axis=(0, 1), keepdims=True)
    tot = jnp.where(i == 0, jnp.zeros_like(ssq), loss_ref[...]) + ssq
    loss_ref[...] = jnp.where(i == NBC - 1, tot * (1.25 / (N_TOK * D)), tot)


def _sc_gather_body(code_hbm, idx_hbm, out_hbm, idx_v, rows_v, sem):
    wid = lax.axis_index("s") * SC_NC + lax.axis_index("c")
    base = wid * SC_BPW
    for c in range(SC_NCH):
        b = base + c * SC_CH
        pltpu.sync_copy(idx_hbm.at[pl.ds(b, SC_CH)], idx_v)
        pltpu.async_copy(code_hbm.at[idx_v], rows_v, sem).wait()
        pltpu.sync_copy(rows_v, out_hbm.at[pl.ds(b, SC_CH)])


@functools.cache
def _sc_gather():
    mesh = plsc.VectorSubcoreMesh(core_axis_name="c", subcore_axis_name="s")
    return pl.kernel(
        _sc_gather_body,
        out_type=jax.ShapeDtypeStruct((N_TOK, D), jnp.float32),
        mesh=mesh,
        scratch_types=[
            pltpu.VMEM((SC_CH,), jnp.int32),
            pltpu.VMEM((SC_CH, D), jnp.float32),
            pltpu.SemaphoreType.DMA,
        ],
    )


def _argmin_call(z_flat, code):
    return pl.pallas_call(
        _argmin_body,
        grid=(NB,),
        in_specs=[
            pl.BlockSpec((BM, D), lambda i: (i, 0)),
            pl.BlockSpec((NE, D), lambda i: (0, 0)),
        ],
        out_specs=pl.BlockSpec((1, 1, BM), lambda i: (i, 0, 0)),
        out_shape=jax.ShapeDtypeStruct((NB, 1, BM), jnp.int32),
        scratch_shapes=[
            pltpu.VMEM((D, NE), jnp.float32),
            pltpu.VMEM((1, NE), jnp.float32),
        ],
    )(z_flat, code)


def _finalize_call(z_flat, gathered):
    return pl.pallas_call(
        _finalize_body,
        grid=(NBC,),
        in_specs=[
            pl.BlockSpec((BMC, D), lambda i: (i, 0)),
            pl.BlockSpec((BMC, D), lambda i: (i, 0)),
        ],
        out_specs=[
            pl.BlockSpec((BMC, D), lambda i: (i, 0)),
            pl.BlockSpec((1, 1), lambda i: (0, 0)),
        ],
        out_shape=[
            jax.ShapeDtypeStruct((N_TOK, D), jnp.float32),
            jax.ShapeDtypeStruct((1, 1), jnp.float32),
        ],
    )(z_flat, gathered)


def kernel(z, code):
    z_flat = z.reshape(N_TOK, D)
    idx3 = _argmin_call(z_flat, code)
    idx = idx3.reshape(N_TOK)
    gathered = _sc_gather()(code, idx)
    zq_flat, loss11 = _finalize_call(z_flat, gathered)
    return (zq_flat.reshape(z.shape), loss11[0, 0], (None, None, idx))


# speedup vs baseline: 1.2963x; 1.2603x over previous
import functools

import jax
import jax.numpy as jnp
from jax import lax
from jax.experimental import pallas as pl
from jax.experimental.pallas import tpu as pltpu
from jax.experimental.pallas import tpu_sc as plsc

N_TOK = 16384
NE = 8192
D = 256
BM = 128
NB = N_TOK // BM
BMC = 512
NBC = N_TOK // BMC

SC_NC = 2
SC_NS = 16
SC_NW = SC_NC * SC_NS
SC_BPW = N_TOK // SC_NW
SC_CH = 128
SC_NCH = SC_BPW // SC_CH


def _argmin_body(z_ref, code_ref, idx_ref, cnt_scr, c2_scr):
    i = pl.program_id(0)

    @pl.when(i == 0)
    def _init():
        c = code_ref[...]
        n = jnp.sqrt(jnp.sum(c * c, axis=1, keepdims=True))
        cn = c / jnp.maximum(n, 1e-12)
        cnt_scr[...] = cn.T
        cnt = cnt_scr[...]
        c2_scr[...] = jnp.sum(cnt * cnt, axis=0, keepdims=True)

    z = z_ref[...]
    zn = z / jnp.maximum(jnp.sqrt(jnp.sum(z * z, axis=1, keepdims=True)), 1e-12)
    z2n = jnp.sum(zn * zn, axis=1, keepdims=True)
    s2 = lax.dot_general(zn * (-2.0), cnt_scr[...], (((1,), (0,)), ((), ())),
                         preferred_element_type=jnp.float32)
    t = (z2n + c2_scr[...]) + s2
    idx_ref[0, 0, :] = jnp.argmin(t, axis=1).astype(jnp.int32)


def _finalize_body(z_ref, g_ref, zq_ref, loss_ref):
    i = pl.program_id(0)
    g = g_ref[...]
    n = jnp.sqrt(jnp.sum(g * g, axis=1, keepdims=True))
    zq = g / jnp.maximum(n, 1e-12)
    zq_ref[...] = zq
    dlt = zq - z_ref[...]
    ssq = jnp.sum(dlt * dlt, axis=(0, 1), keepdims=True)
    tot = jnp.where(i == 0, jnp.zeros_like(ssq), loss_ref[...]) + ssq
    loss_ref[...] = jnp.where(i == NBC - 1, tot * (1.25 / (N_TOK * D)), tot)


def _sc_gather_body(code_hbm, idx_hbm, out_hbm, idx_v, rows_v, sem):
    wid = lax.axis_index("s") * SC_NC + lax.axis_index("c")
    base = wid * SC_BPW
    for c in range(SC_NCH):
        b = base + c * SC_CH
        pltpu.sync_copy(idx_hbm.at[pl.ds(b, SC_CH)], idx_v)
        pltpu.async_copy(code_hbm.at[idx_v], rows_v, sem).wait()
        pltpu.sync_copy(rows_v, out_hbm.at[pl.ds(b, SC_CH)])


@functools.cache
def _sc_gather():
    mesh = plsc.VectorSubcoreMesh(core_axis_name="c", subcore_axis_name="s")
    return pl.kernel(
        _sc_gather_body,
        out_type=jax.ShapeDtypeStruct((N_TOK, D), jnp.float32),
        mesh=mesh,
        scratch_types=[
            pltpu.VMEM((SC_CH,), jnp.int32),
            pltpu.VMEM((SC_CH, D), jnp.float32),
            pltpu.SemaphoreType.DMA,
        ],
    )


def _argmin_call(z_flat, code):
    return pl.pallas_call(
        _argmin_body,
        grid=(NB,),
        in_specs=[
            pl.BlockSpec((BM, D), lambda i: (i, 0)),
            pl.BlockSpec((NE, D), lambda i: (0, 0)),
        ],
        out_specs=pl.BlockSpec((1, 1, BM), lambda i: (i, 0, 0)),
        out_shape=jax.ShapeDtypeStruct((NB, 1, BM), jnp.int32),
        scratch_shapes=[
            pltpu.VMEM((D, NE), jnp.float32),
            pltpu.VMEM((1, NE), jnp.float32),
        ],
    )(z_flat, code)


def _finalize_call(z_flat, gathered):
    return pl.pallas_call(
        _finalize_body,
        grid=(NBC,),
        in_specs=[
            pl.BlockSpec((BMC, D), lambda i: (i, 0)),
            pl.BlockSpec((BMC, D), lambda i: (i, 0)),
        ],
        out_specs=[
            pl.BlockSpec((BMC, D), lambda i: (i, 0)),
            pl.BlockSpec((1, 1), lambda i: (0, 0)),
        ],
        out_shape=[
            jax.ShapeDtypeStruct((N_TOK, D), jnp.float32),
            jax.ShapeDtypeStruct((1, 1), jnp.float32),
        ],
    )(z_flat, gathered)


def kernel(z, code):
    z_flat = z.reshape(N_TOK, D)
    idx3 = _argmin_call(z_flat, code)
    idx = idx3.reshape(N_TOK)
    gathered = _sc_gather()(code, idx)
    zq_flat, loss11 = _finalize_call(z_flat, gathered)
    return (zq_flat.reshape(z.shape), loss11[0, 0], (None, None, idx))


# speedup vs baseline: 1.5102x; 1.1650x over previous
import functools

import jax
import jax.numpy as jnp
from jax import lax
from jax.experimental import pallas as pl
from jax.experimental.pallas import tpu as pltpu
from jax.experimental.pallas import tpu_sc as plsc

N_TOK = 16384
NE = 8192
D = 256
BM = 256
NB = N_TOK // BM
BMC = 512
NBC = N_TOK // BMC

SC_NC = 2
SC_NS = 16
SC_NW = SC_NC * SC_NS
SC_BPW = N_TOK // SC_NW
SC_CH = 128
SC_NCH = SC_BPW // SC_CH


def _argmin_body(z_ref, code_ref, idx_ref, cnt_scr, c2_scr):
    i = pl.program_id(0)

    @pl.when(i == 0)
    def _init():
        c = code_ref[...]
        n = jnp.sqrt(jnp.sum(c * c, axis=1, keepdims=True))
        cn = c / jnp.maximum(n, 1e-12)
        cnt_scr[...] = cn.T
        cnt = cnt_scr[...]
        c2_scr[...] = jnp.sum(cnt * cnt, axis=0, keepdims=True)

    z = z_ref[...]
    zn = z / jnp.maximum(jnp.sqrt(jnp.sum(z * z, axis=1, keepdims=True)), 1e-12)
    z2n = jnp.sum(zn * zn, axis=1, keepdims=True)
    s2 = lax.dot_general(zn * (-2.0), cnt_scr[...], (((1,), (0,)), ((), ())),
                         preferred_element_type=jnp.float32)
    t = (z2n + c2_scr[...]) + s2
    idx_ref[0, 0, :] = jnp.argmin(t, axis=1).astype(jnp.int32)


def _finalize_body(z_ref, g_ref, zq_ref, loss_ref):
    i = pl.program_id(0)
    g = g_ref[...]
    n = jnp.sqrt(jnp.sum(g * g, axis=1, keepdims=True))
    zq = g / jnp.maximum(n, 1e-12)
    zq_ref[...] = zq
    dlt = zq - z_ref[...]
    ssq = jnp.sum(dlt * dlt, axis=(0, 1), keepdims=True)
    tot = jnp.where(i == 0, jnp.zeros_like(ssq), loss_ref[...]) + ssq
    loss_ref[...] = jnp.where(i == NBC - 1, tot * (1.25 / (N_TOK * D)), tot)


def _sc_gather_body(code_hbm, idx_hbm, out_hbm, idx_v, rows_v, sem):
    wid = lax.axis_index("s") * SC_NC + lax.axis_index("c")
    base = wid * SC_BPW
    for c in range(SC_NCH):
        b = base + c * SC_CH
        pltpu.sync_copy(idx_hbm.at[pl.ds(b, SC_CH)], idx_v)
        pltpu.async_copy(code_hbm.at[idx_v], rows_v, sem).wait()
        pltpu.sync_copy(rows_v, out_hbm.at[pl.ds(b, SC_CH)])


@functools.cache
def _sc_gather():
    mesh = plsc.VectorSubcoreMesh(core_axis_name="c", subcore_axis_name="s")
    return pl.kernel(
        _sc_gather_body,
        out_type=jax.ShapeDtypeStruct((N_TOK, D), jnp.float32),
        mesh=mesh,
        scratch_types=[
            pltpu.VMEM((SC_CH,), jnp.int32),
            pltpu.VMEM((SC_CH, D), jnp.float32),
            pltpu.SemaphoreType.DMA,
        ],
    )


def _argmin_call(z_flat, code):
    return pl.pallas_call(
        _argmin_body,
        grid=(NB,),
        in_specs=[
            pl.BlockSpec((BM, D), lambda i: (i, 0)),
            pl.BlockSpec((NE, D), lambda i: (0, 0)),
        ],
        out_specs=pl.BlockSpec((1, 1, BM), lambda i: (i, 0, 0)),
        out_shape=jax.ShapeDtypeStruct((NB, 1, BM), jnp.int32),
        scratch_shapes=[
            pltpu.VMEM((D, NE), jnp.float32),
            pltpu.VMEM((1, NE), jnp.float32),
        ],
    )(z_flat, code)


def _finalize_call(z_flat, gathered):
    return pl.pallas_call(
        _finalize_body,
        grid=(NBC,),
        in_specs=[
            pl.BlockSpec((BMC, D), lambda i: (i, 0)),
            pl.BlockSpec((BMC, D), lambda i: (i, 0)),
        ],
        out_specs=[
            pl.BlockSpec((BMC, D), lambda i: (i, 0)),
            pl.BlockSpec((1, 1), lambda i: (0, 0)),
        ],
        out_shape=[
            jax.ShapeDtypeStruct((N_TOK, D), jnp.float32),
            jax.ShapeDtypeStruct((1, 1), jnp.float32),
        ],
    )(z_flat, gathered)


def kernel(z, code):
    z_flat = z.reshape(N_TOK, D)
    idx3 = _argmin_call(z_flat, code)
    idx = idx3.reshape(N_TOK)
    gathered = _sc_gather()(code, idx)
    zq_flat, loss11 = _finalize_call(z_flat, gathered)
    return (zq_flat.reshape(z.shape), loss11[0, 0], (None, None, idx))


# speedup vs baseline: 1.5650x; 1.0363x over previous
import functools

import jax
import jax.numpy as jnp
from jax import lax
from jax.experimental import pallas as pl
from jax.experimental.pallas import tpu as pltpu
from jax.experimental.pallas import tpu_sc as plsc

N_TOK = 16384
NE = 8192
D = 256
BM = 512
NB = N_TOK // BM
BMC = 512
NBC = N_TOK // BMC

SC_NC = 2
SC_NS = 16
SC_NW = SC_NC * SC_NS
SC_BPW = N_TOK // SC_NW
SC_CH = 128
SC_NCH = SC_BPW // SC_CH


def _prep_body(code_ref, cnt_ref, c2_ref):
    c = code_ref[...]
    n = jnp.sqrt(jnp.sum(c * c, axis=1, keepdims=True))
    cn = c / jnp.maximum(n, 1e-12)
    cnt_ref[...] = cn.T
    cnt = cnt_ref[...]
    c2_ref[...] = jnp.sum(cnt * cnt, axis=0, keepdims=True)


def _argmin_body(z_ref, cnt_ref, c2_ref, idx_ref):
    z = z_ref[...]
    zn = z / jnp.maximum(jnp.sqrt(jnp.sum(z * z, axis=1, keepdims=True)), 1e-12)
    z2n = jnp.sum(zn * zn, axis=1, keepdims=True)
    s2 = lax.dot_general(zn * (-2.0), cnt_ref[...], (((1,), (0,)), ((), ())),
                         preferred_element_type=jnp.float32)
    t = (z2n + c2_ref[...]) + s2
    idx_ref[0, 0, :] = jnp.argmin(t, axis=1).astype(jnp.int32)


def _finalize_body(z_ref, g_ref, zq_ref, loss_ref):
    i = pl.program_id(0)
    g = g_ref[...]
    n = jnp.sqrt(jnp.sum(g * g, axis=1, keepdims=True))
    zq = g / jnp.maximum(n, 1e-12)
    zq_ref[...] = zq
    dlt = zq - z_ref[...]
    ssq = jnp.sum(dlt * dlt, axis=(0, 1), keepdims=True)
    tot = jnp.where(i == 0, jnp.zeros_like(ssq), loss_ref[...]) + ssq
    loss_ref[...] = jnp.where(i == NBC - 1, tot * (1.25 / (N_TOK * D)), tot)


def _sc_gather_body(code_hbm, idx_hbm, out_hbm, idx_v, rows_v, sem):
    wid = lax.axis_index("s") * SC_NC + lax.axis_index("c")
    base = wid * SC_BPW
    for c in range(SC_NCH):
        b = base + c * SC_CH
        pltpu.sync_copy(idx_hbm.at[pl.ds(b, SC_CH)], idx_v)
        pltpu.async_copy(code_hbm.at[idx_v], rows_v, sem).wait()
        pltpu.sync_copy(rows_v, out_hbm.at[pl.ds(b, SC_CH)])


@functools.cache
def _sc_gather():
    mesh = plsc.VectorSubcoreMesh(core_axis_name="c", subcore_axis_name="s")
    return pl.kernel(
        _sc_gather_body,
        out_type=jax.ShapeDtypeStruct((N_TOK, D), jnp.float32),
        mesh=mesh,
        scratch_types=[
            pltpu.VMEM((SC_CH,), jnp.int32),
            pltpu.VMEM((SC_CH, D), jnp.float32),
            pltpu.SemaphoreType.DMA,
        ],
    )


def _prep_call(code):
    return pl.pallas_call(
        _prep_body,
        out_specs=[
            pl.BlockSpec((D, NE), lambda: (0, 0)),
            pl.BlockSpec((1, NE), lambda: (0, 0)),
        ],
        out_shape=[
            jax.ShapeDtypeStruct((D, NE), jnp.float32),
            jax.ShapeDtypeStruct((1, NE), jnp.float32),
        ],
    )(code)


def _argmin_call(z_flat, cnt, c2):
    return pl.pallas_call(
        _argmin_body,
        grid=(NB,),
        in_specs=[
            pl.BlockSpec((BM, D), lambda i: (i, 0)),
            pl.BlockSpec((D, NE), lambda i: (0, 0)),
            pl.BlockSpec((1, NE), lambda i: (0, 0)),
        ],
        out_specs=pl.BlockSpec((1, 1, BM), lambda i: (i, 0, 0)),
        out_shape=jax.ShapeDtypeStruct((NB, 1, BM), jnp.int32),
    )(z_flat, cnt, c2)


def _finalize_call(z_flat, gathered):
    return pl.pallas_call(
        _finalize_body,
        grid=(NBC,),
        in_specs=[
            pl.BlockSpec((BMC, D), lambda i: (i, 0)),
            pl.BlockSpec((BMC, D), lambda i: (i, 0)),
        ],
        out_specs=[
            pl.BlockSpec((BMC, D), lambda i: (i, 0)),
            pl.BlockSpec((1, 1), lambda i: (0, 0)),
        ],
        out_shape=[
            jax.ShapeDtypeStruct((N_TOK, D), jnp.float32),
            jax.ShapeDtypeStruct((1, 1), jnp.float32),
        ],
    )(z_flat, gathered)


def kernel(z, code):
    z_flat = z.reshape(N_TOK, D)
    cnt, c2 = _prep_call(code)
    idx3 = _argmin_call(z_flat, cnt, c2)
    idx = idx3.reshape(N_TOK)
    gathered = _sc_gather()(code, idx)
    zq_flat, loss11 = _finalize_call(z_flat, gathered)
    return (zq_flat.reshape(z.shape), loss11[0, 0], (None, None, idx))
